# baseline (speedup 1.0000x reference)
import functools

import jax
import jax.numpy as jnp
from jax import lax
from jax.experimental import pallas as pl
from jax.experimental.pallas import tpu as pltpu
from jax.experimental.pallas import tpu_sc as plsc

N_VOCAB = 1000000
N_H = 64
BATCH = 4096
SEQ = 50

_info = plsc.get_sparse_core_info()
NC, NS = _info.num_cores, _info.num_subcores
NW = NC * NS
TOTAL = BATCH * SEQ
BPW = TOTAL // NW
CW = 800
NSTEP = BPW // CW

_mesh = plsc.VectorSubcoreMesh(core_axis_name="c", subcore_axis_name="s")


@functools.partial(
    pl.kernel,
    mesh=_mesh,
    out_type=jax.ShapeDtypeStruct((TOTAL, N_H), jnp.float32),
    scratch_types=[
        pltpu.VMEM((BPW,), jnp.int32),
        pltpu.VMEM((CW, N_H), jnp.float32),
        pltpu.VMEM((CW, N_H), jnp.float32),
        pltpu.SemaphoreType.DMA,
        pltpu.SemaphoreType.DMA,
        pltpu.SemaphoreType.DMA,
        pltpu.SemaphoreType.DMA,
    ],
    compiler_params=pltpu.CompilerParams(use_tc_tiling_on_sc=False),
)
def _gather_kernel(table_hbm, idx_hbm, out_hbm, idx_v, rows_a, rows_b,
                   gsem_a, gsem_b, ssem_a, ssem_b):
    wid = lax.axis_index("s") * NC + lax.axis_index("c")
    base = wid * BPW
    rows = (rows_a, rows_b)
    gsem = (gsem_a, gsem_b)
    ssem = (ssem_a, ssem_b)

    pltpu.sync_copy(idx_hbm.at[pl.ds(base, BPW)], idx_v)

    def offs(j):
        return idx_v.at[pl.ds(j * CW, CW)]

    pltpu.async_copy(table_hbm.at[offs(0)], rows[0], gsem[0])

    for j in range(NSTEP):
        b = j & 1
        o = 1 - b
        pltpu.make_async_copy(table_hbm.at[offs(j)], rows[b], gsem[b]).wait()
        if j + 1 < NSTEP:
            if j >= 1:
                pltpu.make_async_copy(rows[o], out_hbm.at[pl.ds(base, CW)],
                                      ssem[o]).wait()
            pltpu.async_copy(table_hbm.at[offs(j + 1)], rows[o], gsem[o])
        pltpu.async_copy(rows[b], out_hbm.at[pl.ds(base + j * CW, CW)],
                         ssem[b])

    pltpu.make_async_copy(rows[0], out_hbm.at[pl.ds(base, CW)],
                          ssem[0]).wait()
    pltpu.make_async_copy(rows[1], out_hbm.at[pl.ds(base, CW)],
                          ssem[1]).wait()


def kernel(inputs, V, b):
    del b
    eye = jnp.eye(N_H, dtype=jnp.float32)
    v_rm = jax.lax.dot_general(V.T, eye, (((0,), (0,)), ((), ())),
                               precision=jax.lax.Precision.HIGHEST)
    idx = inputs.astype(jnp.int32).reshape(TOTAL)
    out2d = _gather_kernel(v_rm, idx)
    return out2d.reshape(BATCH, SEQ, N_H)


# speedup vs baseline: 2.1323x; 2.1323x over previous
import functools

import jax
import jax.numpy as jnp
from jax import lax
from jax.experimental import pallas as pl
from jax.experimental.pallas import tpu as pltpu
from jax.experimental.pallas import tpu_sc as plsc

N_VOCAB = 1000000
N_H = 64
BATCH = 4096
SEQ = 50

_info = plsc.get_sparse_core_info()
NC, NS = _info.num_cores, _info.num_subcores
NW = NC * NS
TOTAL = BATCH * SEQ
BPW = TOTAL // NW
CW = 800
NSTEP = BPW // CW

_mesh = plsc.VectorSubcoreMesh(core_axis_name="c", subcore_axis_name="s")


@functools.partial(
    pl.kernel,
    mesh=_mesh,
    out_type=jax.ShapeDtypeStruct((TOTAL, N_H), jnp.float32),
    scratch_types=[
        pltpu.VMEM((BPW,), jnp.int32),
        pltpu.VMEM((CW, N_H), jnp.float32),
        pltpu.VMEM((CW, N_H), jnp.float32),
        pltpu.SemaphoreType.DMA,
        pltpu.SemaphoreType.DMA,
        pltpu.SemaphoreType.DMA,
        pltpu.SemaphoreType.DMA,
    ],
    compiler_params=pltpu.CompilerParams(use_tc_tiling_on_sc=False),
)
def _gather_kernel(table_hbm, idx_hbm, out_hbm, idx_v, rows_a, rows_b,
                   gsem_a, gsem_b, ssem_a, ssem_b):
    wid = lax.axis_index("s") * NC + lax.axis_index("c")
    base = wid * BPW
    rows = (rows_a, rows_b)
    gsem = (gsem_a, gsem_b)
    ssem = (ssem_a, ssem_b)

    pltpu.sync_copy(idx_hbm.at[pl.ds(base, BPW)], idx_v)

    def offs(j):
        return idx_v.at[pl.ds(j * CW, CW)]

    pltpu.async_copy(table_hbm.at[offs(0)], rows[0], gsem[0])

    for j in range(NSTEP):
        b = j & 1
        o = 1 - b
        pltpu.make_async_copy(table_hbm.at[offs(j)], rows[b], gsem[b]).wait()
        if j + 1 < NSTEP:
            if j >= 1:
                pltpu.make_async_copy(rows[o], out_hbm.at[pl.ds(base, CW)],
                                      ssem[o]).wait()
            pltpu.async_copy(table_hbm.at[offs(j + 1)], rows[o], gsem[o])
        pltpu.async_copy(rows[b], out_hbm.at[pl.ds(base + j * CW, CW)],
                         ssem[b])

    pltpu.make_async_copy(rows[0], out_hbm.at[pl.ds(base, CW)],
                          ssem[0]).wait()
    pltpu.make_async_copy(rows[1], out_hbm.at[pl.ds(base, CW)],
                          ssem[1]).wait()


_TBLK = 4096
_TGRID = 245
_TAIL = 512
_TAIL2 = N_VOCAB - (_TGRID - 1) * _TBLK - _TAIL


def _transpose_body(in_hbm, tail_ref, out_ref, vin0, vin1, sem0, sem1):
    i = pl.program_id(0)

    def start(k, vin, sem):
        @pl.when(k < _TGRID - 1)
        def _():
            off = pl.multiple_of(k * _TBLK, 128)
            pltpu.make_async_copy(in_hbm.at[:, pl.ds(off, _TBLK)], vin,
                                  sem).start()

        @pl.when(k == _TGRID - 1)
        def _():
            pltpu.make_async_copy(
                in_hbm.at[:, pl.ds((_TGRID - 1) * _TBLK, _TAIL)],
                vin.at[:, pl.ds(0, _TAIL)], sem).start()

    def finish(k, vin, sem):
        @pl.when(k < _TGRID - 1)
        def _():
            pltpu.make_async_copy(in_hbm.at[:, pl.ds(0, _TBLK)], vin,
                                  sem).wait()
            x = vin[...]
            h = _TBLK // 2
            z = jnp.concatenate([x[:, :h], x[:, h:]], axis=0)
            out_ref[...] = z.T

        @pl.when(k == _TGRID - 1)
        def _():
            pltpu.make_async_copy(in_hbm.at[:, pl.ds(0, _TAIL)],
                                  vin.at[:, pl.ds(0, _TAIL)], sem).wait()
            xt = jnp.concatenate([vin[:, :_TAIL], tail_ref[...]], axis=1)
            ht = (_TAIL + _TAIL2) // 2
            zt = jnp.concatenate([xt[:, :ht], xt[:, ht:]], axis=0)
            out_ref[pl.ds(0, ht)] = zt.T

    @pl.when(i == 0)
    def _():
        start(i, vin0, sem0)

    even = lax.rem(i, 2) == 0

    @pl.when((i + 1 < _TGRID) & even)
    def _():
        start(i + 1, vin1, sem1)

    @pl.when((i + 1 < _TGRID) & jnp.logical_not(even))
    def _():
        start(i + 1, vin0, sem0)

    @pl.when(even)
    def _():
        finish(i, vin0, sem0)

    @pl.when(jnp.logical_not(even))
    def _():
        finish(i, vin1, sem1)


_tc_transpose = pl.pallas_call(
    _transpose_body,
    grid=(_TGRID,),
    in_specs=[pl.BlockSpec(memory_space=pl.ANY),
              pl.BlockSpec((N_H, _TAIL2), lambda i: (0, 0))],
    out_specs=pl.BlockSpec((_TBLK // 2, 2 * N_H), lambda i: (i, 0)),
    out_shape=jax.ShapeDtypeStruct((N_VOCAB // 2, 2 * N_H), jnp.float32),
    scratch_shapes=[
        pltpu.VMEM((N_H, _TBLK), jnp.float32),
        pltpu.VMEM((N_H, _TBLK), jnp.float32),
        pltpu.SemaphoreType.DMA,
        pltpu.SemaphoreType.DMA,
    ],
)


def kernel(inputs, V, b):
    del b
    v_t = V.T
    tail = jax.lax.slice(v_t, (0, N_VOCAB - _TAIL2), (N_H, N_VOCAB))
    v_rm = _tc_transpose(v_t, tail).reshape(N_VOCAB, N_H)
    v = inputs.astype(jnp.int32).reshape(TOTAL)
    blk_base = 244 * _TBLK
    off = v & (_TBLK - 1)
    h = _TBLK // 2
    m_main = (v & ~(_TBLK - 1)) + jnp.where(
        off < h, off << 1, ((off - h) << 1) + 1)
    toff = v - blk_base
    ht = (_TAIL + _TAIL2) // 2
    m_tail = blk_base + jnp.where(
        toff < ht, toff << 1, ((toff - ht) << 1) + 1)
    idx = jnp.where(v < blk_base, m_main, m_tail)
    out2d = _gather_kernel(v_rm, idx)
    return out2d.reshape(BATCH, SEQ, N_H)
